# baseline (speedup 1.0000x reference)
import functools

import jax
import jax.numpy as jnp
from jax import lax
from jax.experimental import pallas as pl
from jax.experimental.pallas import tpu as pltpu
from jax.experimental.pallas import tpu_sc as plsc

N_NODES_K = 10000
D_IN_K = 128
N_GCNS_K = 4
D_CAT_K = N_GCNS_K * 128
N_EDGES_K = 320000

NC, NS = 2, 16
NW = NC * NS
CHUNK = 128
EPT = 10240
NCH = EPT // CHUNK
E_PAD = NW * EPT
PAD_ROW = N_NODES_K
NP = 10240
ACC_ROWS = NP
ZPT = NP // NS
NSLOT = 8

_mesh = plsc.VectorSubcoreMesh(core_axis_name="c", subcore_axis_name="s")


@functools.partial(
    pl.kernel,
    out_type=jax.ShapeDtypeStruct((NC, NP, 128), jnp.float32),
    mesh=_mesh,
    scratch_types=(
        [pltpu.VMEM((CHUNK, 128), jnp.float32),
         pltpu.VMEM_SHARED((ACC_ROWS, 128), jnp.float32)]
        + [pltpu.VMEM((CHUNK,), jnp.int32) for _ in range(NSLOT)]
        + [pltpu.SemaphoreType.DMA for _ in range(2 * NSLOT)]
    ),
)
def _sc_hist(dst_hbm, ones_hbm, zeros_hbm, deg_hbm, ones_v, acc_sh, *rest):
    idxs = rest[:NSLOT]
    semd = rest[NSLOT:2 * NSLOT]
    semsc = rest[2 * NSLOT:]
    c = lax.axis_index("c")
    s = lax.axis_index("s")
    wid = s * NC + c
    base = wid * NCH
    pltpu.sync_copy(ones_hbm, ones_v)
    pltpu.sync_copy(zeros_hbm.at[pl.ds(s * ZPT, ZPT)], acc_sh.at[pl.ds(s * ZPT, ZPT)])
    plsc.subcore_barrier()

    def group(g, carry):
        j0 = g * NSLOT
        for k in range(NSLOT):
            pltpu.async_copy(dst_hbm.at[base + j0 + k], idxs[k], semd[k])
        for k in range(NSLOT):
            pltpu.make_async_copy(dst_hbm.at[base + j0 + k], idxs[k],
                                  semd[k]).wait()
            pltpu.async_copy(ones_v, acc_sh.at[idxs[k]], semsc[k], add=True)
        for k in range(NSLOT):
            pltpu.make_async_copy(ones_v, acc_sh.at[idxs[k]], semsc[k]).wait()
        return carry

    lax.fori_loop(0, NCH // NSLOT, group, 0)
    plsc.subcore_barrier()
    pltpu.sync_copy(acc_sh.at[pl.ds(s * ZPT, ZPT)], deg_hbm.at[c, pl.ds(s * ZPT, ZPT)])


@functools.partial(
    pl.kernel,
    out_type=jax.ShapeDtypeStruct((NC, NP, D_IN_K), jnp.float32),
    mesh=_mesh,
    scratch_types=[
        pltpu.VMEM((NCH, CHUNK), jnp.int32),
        pltpu.VMEM((CHUNK,), jnp.int32),
        pltpu.VMEM((CHUNK,), jnp.int32),
        pltpu.VMEM((CHUNK, D_IN_K), jnp.float32),
        pltpu.VMEM((CHUNK, D_IN_K), jnp.float32),
        pltpu.VMEM_SHARED((ACC_ROWS, D_IN_K), jnp.float32),
        pltpu.SemaphoreType.DMA,
        pltpu.SemaphoreType.DMA,
        pltpu.SemaphoreType.DMA,
        pltpu.SemaphoreType.DMA,
        pltpu.SemaphoreType.DMA,
        pltpu.SemaphoreType.DMA,
    ],
)
def _sc_agg(src_hbm, dst_hbm, xp_hbm, p_hbm, src_v, idxd0, idxd1, buf0, buf1,
            acc_sh, semg0, semg1, semd0, semd1, semsc0, semsc1):
    c = lax.axis_index("c")
    s = lax.axis_index("s")
    wid = s * NC + c
    base = wid * NCH
    pltpu.sync_copy(src_hbm.at[pl.ds(base, NCH)], src_v)
    pltpu.sync_copy(xp_hbm.at[pl.ds(s * ZPT, ZPT)], acc_sh.at[pl.ds(s * ZPT, ZPT)])
    plsc.subcore_barrier()

    def fire_g(j, buf, sem):
        pltpu.async_copy(xp_hbm.at[src_v.at[j]], buf, sem)

    def wait_g(j, buf, sem):
        pltpu.make_async_copy(xp_hbm.at[src_v.at[j]], buf, sem).wait()

    def fire_d(j, idxd, sem):
        pltpu.async_copy(dst_hbm.at[base + j], idxd, sem)

    def wait_d(j, idxd, sem):
        pltpu.make_async_copy(dst_hbm.at[base + j], idxd, sem).wait()

    def fire_sc(buf, idxd, sem):
        pltpu.async_copy(buf, acc_sh.at[idxd], sem, add=True)

    def wait_sc(buf, idxd, sem):
        pltpu.make_async_copy(buf, acc_sh.at[idxd], sem).wait()

    fire_d(0, idxd0, semd0)
    fire_d(1, idxd1, semd1)
    fire_g(0, buf0, semg0)
    fire_g(1, buf1, semg1)

    def body(jj, carry):
        j0 = 2 * jj
        j1 = j0 + 1
        wait_g(j0, buf0, semg0)
        wait_d(j0, idxd0, semd0)
        fire_sc(buf0, idxd0, semsc0)
        wait_g(j1, buf1, semg1)
        wait_d(j1, idxd1, semd1)
        fire_sc(buf1, idxd1, semsc1)
        wait_sc(buf0, idxd0, semsc0)

        @pl.when(jj < NCH // 2 - 1)
        def _():
            fire_d(j0 + 2, idxd0, semd0)
            fire_g(j0 + 2, buf0, semg0)

        wait_sc(buf1, idxd1, semsc1)

        @pl.when(jj < NCH // 2 - 1)
        def _():
            fire_d(j1 + 2, idxd1, semd1)
            fire_g(j1 + 2, buf1, semg1)

        return carry

    lax.fori_loop(0, NCH // 2, body, 0)
    plsc.subcore_barrier()
    pltpu.sync_copy(acc_sh.at[pl.ds(s * ZPT, ZPT)], p_hbm.at[c, pl.ds(s * ZPT, ZPT)])


_BLK = 1000
_PBLK = 1024


def _tc_prep_body(x_ref, deg_ref, xp_ref, dis_ref):
    deg = deg_ref[0][:, :1] + deg_ref[1][:, :1] + 1.0
    dis = lax.rsqrt(deg)
    dis_ref[...] = dis
    xp_ref[...] = x_ref[...] * dis


def _tc_prep(x, deg2):
    return pl.pallas_call(
        _tc_prep_body,
        grid=(N_NODES_K // _BLK,),
        in_specs=[
            pl.BlockSpec((_BLK, D_IN_K), lambda i: (i, 0)),
            pl.BlockSpec((NC, _BLK, 128), lambda i: (0, i, 0)),
        ],
        out_specs=[
            pl.BlockSpec((_BLK, D_IN_K), lambda i: (i, 0)),
            pl.BlockSpec((_BLK, 1), lambda i: (i, 0)),
        ],
        out_shape=[
            jax.ShapeDtypeStruct((NP, D_IN_K), jnp.float32),
            jax.ShapeDtypeStruct((N_NODES_K, 1), jnp.float32),
        ],
    )(x, deg2)


def _tc_tail_body(p_ref, xp_ref, dis_ref, wc_ref, bc_ref, w1_ref, b1_ref,
                  w2_ref, b2_ref, o_ref):
    bf = jnp.bfloat16
    xa = (p_ref[0] + p_ref[1] - xp_ref[...]) * dis_ref[...]
    h = jnp.tanh(jnp.dot(xa.astype(bf), wc_ref[...],
                         preferred_element_type=jnp.float32) + bc_ref[...])
    y = jnp.tanh(jnp.dot(h.astype(bf), w1_ref[...],
                         preferred_element_type=jnp.float32) + b1_ref[...])
    o_ref[...] = jnp.tanh(jnp.dot(y.astype(bf), w2_ref[...],
                                  preferred_element_type=jnp.float32) + b2_ref[...])


def _tc_tail(p, xp, dis, wc, bc, w1, b1, w2, b2):
    const = lambda i: (0, 0)
    return pl.pallas_call(
        _tc_tail_body,
        grid=(N_NODES_K // _BLK,),
        in_specs=[
            pl.BlockSpec((NC, _BLK, D_IN_K), lambda i: (0, i, 0)),
            pl.BlockSpec((_BLK, D_IN_K), lambda i: (i, 0)),
            pl.BlockSpec((_BLK, 1), lambda i: (i, 0)),
            pl.BlockSpec((D_IN_K, D_CAT_K), const),
            pl.BlockSpec((1, D_CAT_K), const),
            pl.BlockSpec((D_CAT_K, D_CAT_K), const),
            pl.BlockSpec((1, D_CAT_K), const),
            pl.BlockSpec((D_CAT_K, D_CAT_K), const),
            pl.BlockSpec((1, D_CAT_K), const),
        ],
        out_specs=pl.BlockSpec((_BLK, D_CAT_K), lambda i: (i, 0)),
        out_shape=jax.ShapeDtypeStruct((N_NODES_K, D_CAT_K), jnp.float32),
    )(p, xp, dis, wc, bc, w1, b1, w2, b2)


def kernel(x, edge_index, gcn_W, gcn_b, mlp1_W, mlp1_b, mlp2_W, mlp2_b):
    src = edge_index[0]
    dst = edge_index[1]
    npad = E_PAD - N_EDGES_K
    pad_idx = jnp.arange(npad, dtype=jnp.int32)
    pad_src = pad_idx % N_NODES_K
    pad_dst = PAD_ROW + pad_idx % (NP - N_NODES_K)
    src2d = jnp.concatenate([src, pad_src]).reshape(E_PAD // CHUNK, CHUNK)
    dst2d = jnp.concatenate([dst, pad_dst]).reshape(E_PAD // CHUNK, CHUNK)
    ones = jnp.ones((CHUNK, 128), jnp.float32)
    zeros = jnp.zeros((ACC_ROWS, 128), jnp.float32)
    deg2 = _sc_hist(dst2d, ones, zeros)
    xp, dis = _tc_prep(x, deg2)
    p = _sc_agg(src2d, dst2d, xp)

    wc = jnp.transpose(gcn_W, (1, 0, 2)).reshape(D_IN_K, D_CAT_K)
    bc = gcn_b.reshape(1, D_CAT_K)
    return _tc_tail(p, xp, dis, wc.astype(jnp.bfloat16), bc,
                    mlp1_W.astype(jnp.bfloat16), mlp1_b.reshape(1, D_CAT_K),
                    mlp2_W.astype(jnp.bfloat16), mlp2_b.reshape(1, D_CAT_K))


# speedup vs baseline: 1.1471x; 1.1471x over previous
import functools

import jax
import jax.numpy as jnp
from jax import lax
from jax.experimental import pallas as pl
from jax.experimental.pallas import tpu as pltpu
from jax.experimental.pallas import tpu_sc as plsc

N_NODES_K = 10000
D_IN_K = 128
N_GCNS_K = 4
D_CAT_K = N_GCNS_K * 128
N_EDGES_K = 320000

NC, NS = 2, 16
NW = NC * NS
CHUNK = 128
EPT = 10240
NCH = EPT // CHUNK
E_PAD = NW * EPT
PAD_ROW = N_NODES_K
NP = 10240
ACC_ROWS = NP
ZPT = NP // NS
NSLOT = 8

_mesh = plsc.VectorSubcoreMesh(core_axis_name="c", subcore_axis_name="s")


@functools.partial(
    pl.kernel,
    out_type=jax.ShapeDtypeStruct((NC, NP, 128), jnp.float32),
    mesh=_mesh,
    scratch_types=(
        [pltpu.VMEM((CHUNK, 128), jnp.float32),
         pltpu.VMEM_SHARED((ACC_ROWS, 128), jnp.float32)]
        + [pltpu.VMEM((CHUNK,), jnp.int32) for _ in range(NSLOT)]
        + [pltpu.SemaphoreType.DMA for _ in range(2 * NSLOT)]
    ),
)
def _sc_hist(dst_hbm, ones_hbm, zeros_hbm, deg_hbm, ones_v, acc_sh, *rest):
    idxs = rest[:NSLOT]
    semd = rest[NSLOT:2 * NSLOT]
    semsc = rest[2 * NSLOT:]
    c = lax.axis_index("c")
    s = lax.axis_index("s")
    wid = s * NC + c
    base = wid * NCH
    pltpu.sync_copy(ones_hbm, ones_v)
    pltpu.sync_copy(zeros_hbm.at[pl.ds(s * ZPT, ZPT)], acc_sh.at[pl.ds(s * ZPT, ZPT)])
    plsc.subcore_barrier()

    def group(g, carry):
        j0 = g * NSLOT
        for k in range(NSLOT):
            pltpu.async_copy(dst_hbm.at[base + j0 + k], idxs[k], semd[k])
        for k in range(NSLOT):
            pltpu.make_async_copy(dst_hbm.at[base + j0 + k], idxs[k],
                                  semd[k]).wait()
            pltpu.async_copy(ones_v, acc_sh.at[idxs[k]], semsc[k], add=True)
        for k in range(NSLOT):
            pltpu.make_async_copy(ones_v, acc_sh.at[idxs[k]], semsc[k]).wait()
        return carry

    lax.fori_loop(0, NCH // NSLOT, group, 0)
    plsc.subcore_barrier()
    pltpu.sync_copy(acc_sh.at[pl.ds(s * ZPT, ZPT)], deg_hbm.at[c, pl.ds(s * ZPT, ZPT)])


@functools.partial(
    pl.kernel,
    out_type=jax.ShapeDtypeStruct((NC, NP, D_IN_K), jnp.float32),
    mesh=_mesh,
    scratch_types=[
        pltpu.VMEM((NCH, CHUNK), jnp.int32),
        pltpu.VMEM((CHUNK,), jnp.int32),
        pltpu.VMEM((CHUNK,), jnp.int32),
        pltpu.VMEM((CHUNK, D_IN_K), jnp.float32),
        pltpu.VMEM((CHUNK, D_IN_K), jnp.float32),
        pltpu.VMEM_SHARED((ACC_ROWS, D_IN_K), jnp.float32),
        pltpu.SemaphoreType.DMA,
        pltpu.SemaphoreType.DMA,
        pltpu.SemaphoreType.DMA,
        pltpu.SemaphoreType.DMA,
    ],
)
def _sc_agg(src_hbm, dst_hbm, xp_hbm, p_hbm, src_v, idxd0, idxd1, buf0, buf1,
            acc_sh, semg0, semg1, semd0, semd1):
    c = lax.axis_index("c")
    s = lax.axis_index("s")
    wid = s * NC + c
    base = wid * NCH
    pltpu.sync_copy(src_hbm.at[pl.ds(base, NCH)], src_v)
    pltpu.sync_copy(xp_hbm.at[pl.ds(s * ZPT, ZPT)], acc_sh.at[pl.ds(s * ZPT, ZPT)])
    plsc.subcore_barrier()

    def fire_g(j, buf, sem):
        pltpu.async_copy(xp_hbm.at[src_v.at[j]], buf, sem)

    def wait_g(j, buf, sem):
        pltpu.make_async_copy(xp_hbm.at[src_v.at[j]], buf, sem).wait()

    def fire_d(j, idxd, sem):
        pltpu.async_copy(dst_hbm.at[base + j], idxd, sem)

    def wait_d(j, idxd, sem):
        pltpu.make_async_copy(dst_hbm.at[base + j], idxd, sem).wait()

    def scat(buf, idxd):
        pltpu.sync_copy(buf, acc_sh.at[idxd], add=True)

    fire_d(0, idxd0, semd0)
    fire_d(1, idxd1, semd1)
    fire_g(0, buf0, semg0)

    def body(jj, carry):
        j0 = 2 * jj
        j1 = j0 + 1
        fire_g(j1, buf1, semg1)
        wait_g(j0, buf0, semg0)
        wait_d(j0, idxd0, semd0)
        scat(buf0, idxd0)

        @pl.when(jj < NCH // 2 - 1)
        def _():
            fire_d(j0 + 2, idxd0, semd0)
            fire_g(j0 + 2, buf0, semg0)

        wait_g(j1, buf1, semg1)
        wait_d(j1, idxd1, semd1)
        scat(buf1, idxd1)

        @pl.when(jj < NCH // 2 - 1)
        def _():
            fire_d(j1 + 2, idxd1, semd1)

        return carry

    lax.fori_loop(0, NCH // 2, body, 0)
    plsc.subcore_barrier()
    pltpu.sync_copy(acc_sh.at[pl.ds(s * ZPT, ZPT)], p_hbm.at[c, pl.ds(s * ZPT, ZPT)])


_BLK = 1000
_PBLK = 1024


def _tc_prep_body(x_ref, deg_ref, xp_ref, dis_ref):
    deg = deg_ref[0][:, :1] + deg_ref[1][:, :1] + 1.0
    dis = lax.rsqrt(deg)
    dis_ref[...] = dis
    xp_ref[...] = x_ref[...] * dis


def _tc_prep(x, deg2):
    return pl.pallas_call(
        _tc_prep_body,
        grid=(N_NODES_K // _BLK,),
        in_specs=[
            pl.BlockSpec((_BLK, D_IN_K), lambda i: (i, 0)),
            pl.BlockSpec((NC, _BLK, 128), lambda i: (0, i, 0)),
        ],
        out_specs=[
            pl.BlockSpec((_BLK, D_IN_K), lambda i: (i, 0)),
            pl.BlockSpec((_BLK, 1), lambda i: (i, 0)),
        ],
        out_shape=[
            jax.ShapeDtypeStruct((NP, D_IN_K), jnp.float32),
            jax.ShapeDtypeStruct((N_NODES_K, 1), jnp.float32),
        ],
    )(x, deg2)


def _tc_tail_body(p_ref, xp_ref, dis_ref, wc_ref, bc_ref, w1_ref, b1_ref,
                  w2_ref, b2_ref, o_ref):
    bf = jnp.bfloat16
    xa = (p_ref[0] + p_ref[1] - xp_ref[...]) * dis_ref[...]
    h = jnp.tanh(jnp.dot(xa.astype(bf), wc_ref[...],
                         preferred_element_type=jnp.float32) + bc_ref[...])
    y = jnp.tanh(jnp.dot(h.astype(bf), w1_ref[...],
                         preferred_element_type=jnp.float32) + b1_ref[...])
    o_ref[...] = jnp.tanh(jnp.dot(y.astype(bf), w2_ref[...],
                                  preferred_element_type=jnp.float32) + b2_ref[...])


def _tc_tail(p, xp, dis, wc, bc, w1, b1, w2, b2):
    const = lambda i: (0, 0)
    return pl.pallas_call(
        _tc_tail_body,
        grid=(N_NODES_K // _BLK,),
        in_specs=[
            pl.BlockSpec((NC, _BLK, D_IN_K), lambda i: (0, i, 0)),
            pl.BlockSpec((_BLK, D_IN_K), lambda i: (i, 0)),
            pl.BlockSpec((_BLK, 1), lambda i: (i, 0)),
            pl.BlockSpec((D_IN_K, D_CAT_K), const),
            pl.BlockSpec((1, D_CAT_K), const),
            pl.BlockSpec((D_CAT_K, D_CAT_K), const),
            pl.BlockSpec((1, D_CAT_K), const),
            pl.BlockSpec((D_CAT_K, D_CAT_K), const),
            pl.BlockSpec((1, D_CAT_K), const),
        ],
        out_specs=pl.BlockSpec((_BLK, D_CAT_K), lambda i: (i, 0)),
        out_shape=jax.ShapeDtypeStruct((N_NODES_K, D_CAT_K), jnp.float32),
    )(p, xp, dis, wc, bc, w1, b1, w2, b2)


def kernel(x, edge_index, gcn_W, gcn_b, mlp1_W, mlp1_b, mlp2_W, mlp2_b):
    src = edge_index[0]
    dst = edge_index[1]
    npad = E_PAD - N_EDGES_K
    pad_idx = jnp.arange(npad, dtype=jnp.int32)
    pad_src = pad_idx % N_NODES_K
    pad_dst = PAD_ROW + pad_idx % (NP - N_NODES_K)
    src2d = jnp.concatenate([src, pad_src]).reshape(E_PAD // CHUNK, CHUNK)
    dst2d = jnp.concatenate([dst, pad_dst]).reshape(E_PAD // CHUNK, CHUNK)
    ones = jnp.ones((CHUNK, 128), jnp.float32)
    zeros = jnp.zeros((ACC_ROWS, 128), jnp.float32)
    deg2 = _sc_hist(dst2d, ones, zeros)
    xp, dis = _tc_prep(x, deg2)
    p = _sc_agg(src2d, dst2d, xp)

    wc = jnp.transpose(gcn_W, (1, 0, 2)).reshape(D_IN_K, D_CAT_K)
    bc = gcn_b.reshape(1, D_CAT_K)
    return _tc_tail(p, xp, dis, wc.astype(jnp.bfloat16), bc,
                    mlp1_W.astype(jnp.bfloat16), mlp1_b.reshape(1, D_CAT_K),
                    mlp2_W.astype(jnp.bfloat16), mlp2_b.reshape(1, D_CAT_K))


# speedup vs baseline: 1.2106x; 1.0553x over previous
import functools

import jax
import jax.numpy as jnp
from jax import lax
from jax.experimental import pallas as pl
from jax.experimental.pallas import tpu as pltpu
from jax.experimental.pallas import tpu_sc as plsc

N_NODES_K = 10000
D_IN_K = 128
N_GCNS_K = 4
D_CAT_K = N_GCNS_K * 128
N_EDGES_K = 320000

NC, NS = 2, 16
NW = NC * NS
CHUNK = 128
NCHT = 78
NCH_ALL = N_EDGES_K // CHUNK
NP = 10240
ACC_ROWS = NP
ZPT = NP // NS
NSLOT = 6

_mesh = plsc.VectorSubcoreMesh(core_axis_name="c", subcore_axis_name="s")


@functools.partial(
    pl.kernel,
    out_type=jax.ShapeDtypeStruct((NC, NP, 128), jnp.float32),
    mesh=_mesh,
    scratch_types=(
        [pltpu.VMEM((CHUNK, 128), jnp.float32),
         pltpu.VMEM_SHARED((ACC_ROWS, 128), jnp.float32)]
        + [pltpu.VMEM((CHUNK,), jnp.int32) for _ in range(NSLOT)]
        + [pltpu.SemaphoreType.DMA for _ in range(2 * NSLOT)]
    ),
)
def _sc_hist(e_hbm, ones_hbm, zeros_hbm, deg_hbm, ones_v, acc_sh, *rest):
    idxs = rest[:NSLOT]
    semd = rest[NSLOT:2 * NSLOT]
    semsc = rest[2 * NSLOT:]
    c = lax.axis_index("c")
    s = lax.axis_index("s")
    wid = s * NC + c
    base_e = (wid * NCHT + jnp.minimum(wid, 4)) * CHUNK
    pltpu.sync_copy(ones_hbm, ones_v)
    pltpu.sync_copy(zeros_hbm.at[pl.ds(s * ZPT, ZPT)], acc_sh.at[pl.ds(s * ZPT, ZPT)])
    plsc.subcore_barrier()

    def dst_at(j):
        return e_hbm.at[1, pl.ds(base_e + j * CHUNK, CHUNK)]

    def group(g, carry):
        j0 = g * NSLOT
        for k in range(NSLOT):
            pltpu.async_copy(dst_at(j0 + k), idxs[k], semd[k])
        for k in range(NSLOT):
            pltpu.make_async_copy(dst_at(j0 + k), idxs[k], semd[k]).wait()
            pltpu.async_copy(ones_v, acc_sh.at[idxs[k]], semsc[k], add=True)
        for k in range(NSLOT):
            pltpu.make_async_copy(ones_v, acc_sh.at[idxs[k]], semsc[k]).wait()
        return carry

    lax.fori_loop(0, NCHT // NSLOT, group, 0)

    @pl.when(wid < 4)
    def _():
        pltpu.sync_copy(dst_at(NCHT), idxs[0])
        pltpu.sync_copy(ones_v, acc_sh.at[idxs[0]], add=True)

    plsc.subcore_barrier()
    pltpu.sync_copy(acc_sh.at[pl.ds(s * ZPT, ZPT)], deg_hbm.at[c, pl.ds(s * ZPT, ZPT)])


@functools.partial(
    pl.kernel,
    out_type=jax.ShapeDtypeStruct((NC, NP, D_IN_K), jnp.float32),
    mesh=_mesh,
    scratch_types=[
        pltpu.VMEM(((NCHT + 1) * CHUNK,), jnp.int32),
        pltpu.VMEM((CHUNK,), jnp.int32),
        pltpu.VMEM((CHUNK,), jnp.int32),
        pltpu.VMEM((CHUNK, D_IN_K), jnp.float32),
        pltpu.VMEM((CHUNK, D_IN_K), jnp.float32),
        pltpu.VMEM_SHARED((ACC_ROWS, D_IN_K), jnp.float32),
        pltpu.SemaphoreType.DMA,
        pltpu.SemaphoreType.DMA,
        pltpu.SemaphoreType.DMA,
        pltpu.SemaphoreType.DMA,
    ],
)
def _sc_agg(e_hbm, xp_hbm, p_hbm, src_v, idxd0, idxd1, buf0, buf1,
            acc_sh, semg0, semg1, semd0, semd1):
    c = lax.axis_index("c")
    s = lax.axis_index("s")
    wid = s * NC + c
    base_e = (wid * NCHT + jnp.minimum(wid, 4)) * CHUNK
    pltpu.sync_copy(e_hbm.at[0, pl.ds(base_e, NCHT * CHUNK)],
                    src_v.at[pl.ds(0, NCHT * CHUNK)])
    pltpu.sync_copy(xp_hbm.at[pl.ds(s * ZPT, ZPT)], acc_sh.at[pl.ds(s * ZPT, ZPT)])
    plsc.subcore_barrier()

    def fire_g(j, buf, sem):
        pltpu.async_copy(xp_hbm.at[src_v.at[pl.ds(j * CHUNK, CHUNK)]], buf, sem)

    def wait_g(j, buf, sem):
        pltpu.make_async_copy(xp_hbm.at[src_v.at[pl.ds(j * CHUNK, CHUNK)]],
                              buf, sem).wait()

    def fire_d(j, idxd, sem):
        pltpu.async_copy(e_hbm.at[1, pl.ds(base_e + j * CHUNK, CHUNK)], idxd, sem)

    def wait_d(j, idxd, sem):
        pltpu.make_async_copy(e_hbm.at[1, pl.ds(base_e + j * CHUNK, CHUNK)],
                              idxd, sem).wait()

    def scat(buf, idxd):
        pltpu.sync_copy(buf, acc_sh.at[idxd], add=True)

    fire_d(0, idxd0, semd0)
    fire_d(1, idxd1, semd1)
    fire_g(0, buf0, semg0)

    def body(jj, carry):
        j0 = 2 * jj
        j1 = j0 + 1
        fire_g(j1, buf1, semg1)
        wait_g(j0, buf0, semg0)
        wait_d(j0, idxd0, semd0)
        scat(buf0, idxd0)

        @pl.when(jj < NCHT // 2 - 1)
        def _():
            fire_d(j0 + 2, idxd0, semd0)
            fire_g(j0 + 2, buf0, semg0)

        wait_g(j1, buf1, semg1)
        wait_d(j1, idxd1, semd1)
        scat(buf1, idxd1)

        @pl.when(jj < NCHT // 2 - 1)
        def _():
            fire_d(j1 + 2, idxd1, semd1)

        return carry

    lax.fori_loop(0, NCHT // 2, body, 0)

    @pl.when(wid < 4)
    def _():
        pltpu.sync_copy(e_hbm.at[0, pl.ds(base_e + NCHT * CHUNK, CHUNK)],
                        src_v.at[pl.ds(NCHT * CHUNK, CHUNK)])
        pltpu.sync_copy(e_hbm.at[1, pl.ds(base_e + NCHT * CHUNK, CHUNK)], idxd0)
        pltpu.async_copy(xp_hbm.at[src_v.at[pl.ds(NCHT * CHUNK, CHUNK)]],
                         buf0, semg0).wait()
        pltpu.sync_copy(buf0, acc_sh.at[idxd0], add=True)

    plsc.subcore_barrier()
    pltpu.sync_copy(acc_sh.at[pl.ds(s * ZPT, ZPT)], p_hbm.at[c, pl.ds(s * ZPT, ZPT)])


_BLK = 1000
_PBLK = 1024


def _tc_prep_body(x_ref, deg_ref, xp_ref, dis_ref):
    deg = deg_ref[0][:, :1] + deg_ref[1][:, :1] + 1.0
    dis = lax.rsqrt(deg)
    dis_ref[...] = dis
    xp_ref[...] = x_ref[...] * dis


def _tc_prep(x, deg2):
    return pl.pallas_call(
        _tc_prep_body,
        grid=(N_NODES_K // _BLK,),
        in_specs=[
            pl.BlockSpec((_BLK, D_IN_K), lambda i: (i, 0)),
            pl.BlockSpec((NC, _BLK, 128), lambda i: (0, i, 0)),
        ],
        out_specs=[
            pl.BlockSpec((_BLK, D_IN_K), lambda i: (i, 0)),
            pl.BlockSpec((_BLK, 1), lambda i: (i, 0)),
        ],
        out_shape=[
            jax.ShapeDtypeStruct((NP, D_IN_K), jnp.float32),
            jax.ShapeDtypeStruct((N_NODES_K, 1), jnp.float32),
        ],
    )(x, deg2)


def _tc_tail_body(p_ref, xp_ref, dis_ref, wc_ref, bc_ref, w1_ref, b1_ref,
                  w2_ref, b2_ref, o_ref):
    bf = jnp.bfloat16
    xa = (p_ref[0] + p_ref[1] - xp_ref[...]) * dis_ref[...]
    h = jnp.tanh(jnp.dot(xa.astype(bf), wc_ref[...],
                         preferred_element_type=jnp.float32) + bc_ref[...])
    y = jnp.tanh(jnp.dot(h.astype(bf), w1_ref[...],
                         preferred_element_type=jnp.float32) + b1_ref[...])
    o_ref[...] = jnp.tanh(jnp.dot(y.astype(bf), w2_ref[...],
                                  preferred_element_type=jnp.float32) + b2_ref[...])


def _tc_tail(p, xp, dis, wc, bc, w1, b1, w2, b2):
    const = lambda i: (0, 0)
    return pl.pallas_call(
        _tc_tail_body,
        grid=(N_NODES_K // _BLK,),
        in_specs=[
            pl.BlockSpec((NC, _BLK, D_IN_K), lambda i: (0, i, 0)),
            pl.BlockSpec((_BLK, D_IN_K), lambda i: (i, 0)),
            pl.BlockSpec((_BLK, 1), lambda i: (i, 0)),
            pl.BlockSpec((D_IN_K, D_CAT_K), const),
            pl.BlockSpec((1, D_CAT_K), const),
            pl.BlockSpec((D_CAT_K, D_CAT_K), const),
            pl.BlockSpec((1, D_CAT_K), const),
            pl.BlockSpec((D_CAT_K, D_CAT_K), const),
            pl.BlockSpec((1, D_CAT_K), const),
        ],
        out_specs=pl.BlockSpec((_BLK, D_CAT_K), lambda i: (i, 0)),
        out_shape=jax.ShapeDtypeStruct((N_NODES_K, D_CAT_K), jnp.float32),
    )(p, xp, dis, wc, bc, w1, b1, w2, b2)


def kernel(x, edge_index, gcn_W, gcn_b, mlp1_W, mlp1_b, mlp2_W, mlp2_b):
    ones = jnp.ones((CHUNK, 128), jnp.float32)
    zeros = jnp.zeros((ACC_ROWS, 128), jnp.float32)
    deg2 = _sc_hist(edge_index, ones, zeros)
    xp, dis = _tc_prep(x, deg2)
    p = _sc_agg(edge_index, xp)

    wc = jnp.transpose(gcn_W, (1, 0, 2)).reshape(D_IN_K, D_CAT_K)
    bc = gcn_b.reshape(1, D_CAT_K)
    return _tc_tail(p, xp, dis, wc.astype(jnp.bfloat16), bc,
                    mlp1_W.astype(jnp.bfloat16), mlp1_b.reshape(1, D_CAT_K),
                    mlp2_W.astype(jnp.bfloat16), mlp2_b.reshape(1, D_CAT_K))
